# baseline (speedup 1.0000x reference)
import functools

import jax
import jax.numpy as jnp
from jax import lax
from jax.experimental import pallas as pl
from jax.experimental.pallas import tpu as pltpu
from jax.experimental.pallas import tpu_sc as plsc

BATCH = 16384
D = 64
NC = 2
NS = 16
NW = NC * NS
BPW = BATCH // NW
L = 16
CH = 128
NCH = BPW // CH
MV_BR = 8192


def _mv_body(t_ref, w_ref, oe_ref, oo_ref):
    res = jax.lax.dot_general(
        w_ref[...], t_ref[...],
        (((1,), (1,)), ((), ())),
        preferred_element_type=jnp.float32,
    )
    oe_ref[...] = res[0]
    oo_ref[...] = res[8]


def _matvec(table2, w16):
    n = table2.shape[0]
    grid = pl.cdiv(n, MV_BR)
    return pl.pallas_call(
        _mv_body,
        grid=(grid,),
        in_specs=[
            pl.BlockSpec((MV_BR, 2 * D), lambda i: (i, 0)),
            pl.BlockSpec((16, 2 * D), lambda i: (0, 0)),
        ],
        out_specs=[
            pl.BlockSpec((MV_BR,), lambda i: (i,)),
            pl.BlockSpec((MV_BR,), lambda i: (i,)),
        ],
        out_shape=[
            jax.ShapeDtypeStruct((n,), jnp.float32),
            jax.ShapeDtypeStruct((n,), jnp.float32),
        ],
    )(table2, w16)


def _gather_body(uid_hbm, mid_hbm, sue_hbm, suo_hbm, sme_hbm, smo_hbm,
                 bb_hbm, out_hbm,
                 uidx, midx, utile, mtile, sue, suo, sme, smo, bv, outv,
                 usem, msem):
    wid = lax.axis_index("s") * NC + lax.axis_index("c")
    base = wid * BPW

    pltpu.sync_copy(uid_hbm.at[pl.ds(base, BPW)], uidx)
    pltpu.sync_copy(mid_hbm.at[pl.ds(base, BPW)], midx)
    pltpu.sync_copy(bb_hbm, bv)

    for j in range(BPW // L):
        sl = pl.ds(j * L, L)
        utile[sl] = uidx[sl] >> 1
        mtile[sl] = midx[sl] >> 1

    copies = []
    for j in range(NCH):
        sl = pl.ds(j * CH, CH)
        copies.append(pltpu.async_copy(
            sue_hbm.at[utile.at[sl]], sue.at[sl], usem))
        copies.append(pltpu.async_copy(
            suo_hbm.at[utile.at[sl]], suo.at[sl], usem))
        copies.append(pltpu.async_copy(
            sme_hbm.at[mtile.at[sl]], sme.at[sl], msem))
        copies.append(pltpu.async_copy(
            smo_hbm.at[mtile.at[sl]], smo.at[sl], msem))
    for c in copies:
        c.wait()

    bvec = bv[...]
    for j in range(BPW // L):
        sl = pl.ds(j * L, L)
        uodd = (uidx[sl] & 1) == 1
        modd = (midx[sl] & 1) == 1
        us = jnp.where(uodd, suo[sl], sue[sl])
        ms = jnp.where(modd, smo[sl], sme[sl])
        outv[sl] = us + ms + bvec

    pltpu.sync_copy(outv, out_hbm.at[pl.ds(base, BPW)])


@jax.jit
def _cf_call(user_ids, movie_ids, ut2, mt2, w16u, w16m, bb):
    sue, suo = _matvec(ut2, w16u)
    sme, smo = _matvec(mt2, w16m)
    mesh = plsc.VectorSubcoreMesh(core_axis_name="c", subcore_axis_name="s")
    f = functools.partial(
        pl.kernel,
        mesh=mesh,
        compiler_params=pltpu.CompilerParams(
            needs_layout_passes=False, use_tc_tiling_on_sc=False
        ),
        out_type=jax.ShapeDtypeStruct((BATCH,), jnp.float32),
        scratch_types=[
            pltpu.VMEM((BPW,), jnp.int32),
            pltpu.VMEM((BPW,), jnp.int32),
            pltpu.VMEM((BPW,), jnp.int32),
            pltpu.VMEM((BPW,), jnp.int32),
            pltpu.VMEM((BPW,), jnp.float32),
            pltpu.VMEM((BPW,), jnp.float32),
            pltpu.VMEM((BPW,), jnp.float32),
            pltpu.VMEM((BPW,), jnp.float32),
            pltpu.VMEM((L,), jnp.float32),
            pltpu.VMEM((BPW,), jnp.float32),
            pltpu.SemaphoreType.DMA,
            pltpu.SemaphoreType.DMA,
        ],
    )(_gather_body)
    return f(user_ids, movie_ids, sue, suo, sme, smo, bb)


def _w16(w):
    zero = jnp.zeros((D,), jnp.float32)
    lo = jnp.concatenate([w, zero])
    hi = jnp.concatenate([zero, w])
    return jnp.stack([lo] * 8 + [hi] * 8)


def kernel(user_ids, movie_ids, user_table, movie_table, W, b):
    bb = jnp.broadcast_to(b.reshape(1), (L,))
    return _cf_call(
        user_ids.astype(jnp.int32), movie_ids.astype(jnp.int32),
        user_table.reshape(-1, 2 * D), movie_table.reshape(-1, 2 * D),
        _w16(W[:D, 0]), _w16(W[D:, 0]), bb,
    )


# speedup vs baseline: 1.2578x; 1.2578x over previous
import functools

import jax
import jax.numpy as jnp
from jax import lax
from jax.experimental import pallas as pl
from jax.experimental.pallas import tpu as pltpu
from jax.experimental.pallas import tpu_sc as plsc

BATCH = 16384
D = 64
NC = 2
NS = 16
NW = NC * NS
BPW = BATCH // NW
L = 16
CH = 128
NCH = BPW // CH
MV_BR = 8192


def _mv_manual(table, w8, start, rows, br):
    nb = rows // br
    assert nb * br == rows

    def body(t_hbm, w_ref, o_ref, bufs, sems):
        i = pl.program_id(0)
        par = lax.rem(i, 2)

        def copy_in(blk, buf):
            return pltpu.make_async_copy(
                t_hbm.at[pl.ds(start + blk * br, br), :],
                bufs.at[buf], sems.at[buf])

        @pl.when(i == 0)
        def _():
            copy_in(i, par).start()

        @pl.when(i + 1 < nb)
        def _():
            copy_in(i + 1, lax.rem(i + 1, 2)).start()

        copy_in(i, par).wait()
        res = lax.dot_general(
            w_ref[...], bufs[par],
            (((1,), (1,)), ((), ())),
            preferred_element_type=jnp.float32,
        )
        o_ref[...] = res[0]

    return pl.pallas_call(
        body,
        grid=(nb,),
        in_specs=[
            pl.BlockSpec(memory_space=pl.ANY),
            pl.BlockSpec((8, D), lambda i: (0, 0)),
        ],
        out_specs=pl.BlockSpec((br,), lambda i: (i,)),
        out_shape=jax.ShapeDtypeStruct((rows,), jnp.float32),
        scratch_shapes=[
            pltpu.VMEM((2, br, D), jnp.float32),
            pltpu.SemaphoreType.DMA((2,)),
        ],
    )(table, w8)


def _scores(table, w8):
    n = table.shape[0]
    nb = n // MV_BR
    main_rows = nb * MV_BR
    main = _mv_manual(table, w8, 0, main_rows, MV_BR)
    if main_rows == n:
        return main
    tail_rows = -(-(n - main_rows) // 1024) * 1024
    tail_start = n - tail_rows
    assert tail_start % 8 == 0
    tail = _mv_manual(table, w8, tail_start, tail_rows, tail_rows)
    return jnp.concatenate([main[:tail_start], tail])


def _gather_body(uid_hbm, mid_hbm, su_hbm, sm_hbm, bb_hbm, out_hbm,
                 uidx, midx, sug, smg, bv, outv, usem, msem):
    wid = lax.axis_index("s") * NC + lax.axis_index("c")
    base = wid * BPW

    pltpu.sync_copy(uid_hbm.at[pl.ds(base, BPW)], uidx)
    pltpu.sync_copy(mid_hbm.at[pl.ds(base, BPW)], midx)
    pltpu.sync_copy(bb_hbm, bv)

    copies = []
    for j in range(NCH):
        sl = pl.ds(j * CH, CH)
        copies.append(pltpu.async_copy(su_hbm.at[uidx.at[sl]], sug.at[sl], usem))
        copies.append(pltpu.async_copy(sm_hbm.at[midx.at[sl]], smg.at[sl], msem))
    for c in copies:
        c.wait()

    bvec = bv[...]
    for j in range(BPW // L):
        sl = pl.ds(j * L, L)
        outv[sl] = sug[sl] + smg[sl] + bvec

    pltpu.sync_copy(outv, out_hbm.at[pl.ds(base, BPW)])


@jax.jit
def _cf_call(user_ids, movie_ids, user_table, movie_table, w8u, w8m, bb):
    su = _scores(user_table, w8u)
    sm = _scores(movie_table, w8m)
    mesh = plsc.VectorSubcoreMesh(core_axis_name="c", subcore_axis_name="s")
    f = functools.partial(
        pl.kernel,
        mesh=mesh,
        compiler_params=pltpu.CompilerParams(
            needs_layout_passes=False, use_tc_tiling_on_sc=False
        ),
        out_type=jax.ShapeDtypeStruct((BATCH,), jnp.float32),
        scratch_types=[
            pltpu.VMEM((BPW,), jnp.int32),
            pltpu.VMEM((BPW,), jnp.int32),
            pltpu.VMEM((BPW,), jnp.float32),
            pltpu.VMEM((BPW,), jnp.float32),
            pltpu.VMEM((L,), jnp.float32),
            pltpu.VMEM((BPW,), jnp.float32),
            pltpu.SemaphoreType.DMA,
            pltpu.SemaphoreType.DMA,
        ],
    )(_gather_body)
    return f(user_ids, movie_ids, su, sm, bb)


def kernel(user_ids, movie_ids, user_table, movie_table, W, b):
    w8u = jnp.broadcast_to(W[:D, 0], (8, D))
    w8m = jnp.broadcast_to(W[D:, 0], (8, D))
    bb = jnp.broadcast_to(b.reshape(1), (L,))
    return _cf_call(
        user_ids.astype(jnp.int32), movie_ids.astype(jnp.int32),
        user_table, movie_table, w8u, w8m, bb,
    )
